# baseline (speedup 1.0000x reference)
import functools

import jax
import jax.numpy as jnp
from jax import lax
from jax.experimental import pallas as pl
from jax.experimental.pallas import tpu as pltpu
from jax.experimental.pallas import tpu_sc as plsc

B = 4096
V = 100001
D = 32
CW = 3200
TAIL0 = 99968
LASTLO = 96768
OUTROWS = B + 128
PACK = 4096
HALF = 2048
MAXB = HALF // 128 + 1


def _make_sc_gather(num_cores, num_subcores):
    NW = num_cores * num_subcores
    assert (NW - 1) * CW < TAIL0 <= LASTLO + CW
    mesh = plsc.VectorSubcoreMesh(core_axis_name="c", subcore_axis_name="s")

    @functools.partial(
        pl.kernel,
        mesh=mesh,
        compiler_params=pltpu.CompilerParams(
            use_tc_tiling_on_sc=True, needs_layout_passes=False),
        out_type=[
            jax.ShapeDtypeStruct((OUTROWS, 128), jnp.float32),
            jax.ShapeDtypeStruct((OUTROWS, 128), jnp.float32),
        ],
        scratch_types=[
            pltpu.VMEM((D, CW), jnp.float32),
            pltpu.VMEM((B,), jnp.int32),
            pltpu.VMEM((B,), jnp.int32),
            pltpu.VMEM((HALF + 144,), jnp.int32),
            pltpu.VMEM((128,), jnp.int32),
            pltpu.VMEM((128, 128), jnp.float32),
            pltpu.SemaphoreType.DMA,
            pltpu.SemaphoreType.DMA,
            pltpu.SemaphoreType.DMA,
        ],
    )
    def sc_gather(uid_h, pid_h, ut_h, pt_h, tu_h, tp_h, ou_h, op_h,
                  chunk, uidx, pidx, cval, cpos, staging,
                  csem, isem, ssem):
        cix = lax.axis_index("c")
        six = lax.axis_index("s")
        wid = six * num_cores + cix
        lo = wid * CW
        col_lo = jnp.minimum(lo, LASTLO)
        dma_lo = pl.multiple_of(col_lo, 128)

        iota16 = lax.iota(jnp.int32, 16)
        posP = iota16 * PACK
        trash16 = 4096 + ((iota16 * 8 + wid) & 127)
        trash_packed = trash16 * PACK

        icp_u = pltpu.async_copy(uid_h, uidx, isem)
        icp_p = pltpu.async_copy(pid_h, pidx, isem)
        icp_u.wait()
        icp_p.wait()

        def run_batches(out_hbm, cnt):
            def batch(b, _):
                @pl.when(b * 128 < cnt)
                def _():
                    def per_group(g, _):
                        val16 = cval[pl.ds(b * 128 + g * 16, 16)]
                        pos16 = lax.shift_right_logical(val16, 12)
                        cpos[pl.ds(g * 16, 16)] = pos16
                        q16 = val16 & (PACK - 1)
                        for e in range(16):
                            qv = jnp.broadcast_to(q16[e], (16,))
                            r0 = plsc.load_gather(chunk, [iota16, qv])
                            r1 = plsc.load_gather(chunk, [iota16 + 16, qv])
                            staging[g * 16 + e, pl.ds(0, 16)] = r0
                            staging[g * 16 + e, pl.ds(16, 16)] = r1
                        return 0

                    lax.fori_loop(0, 8, per_group, 0)
                    pltpu.async_copy(staging, out_hbm.at[cpos], ssem).wait()

                return 0

            lax.fori_loop(0, MAXB, batch, 0)

        def scan_round(idx_ref, pos0, in_lo, in_hi, sub):
            def scan_step(j, cnt):
                v16 = idx_ref[pl.ds(pos0 + j * 16, 16)]
                m = (v16 >= in_lo) & (v16 < in_hi)
                pc = plsc.all_reduce_population_count(m)[0]

                @pl.when(pc > 0)
                def _():
                    key = 1 - m.astype(jnp.int32)
                    val = jnp.where(
                        m, posP + (pos0 + j * 16) * PACK + (v16 - sub),
                        trash_packed)
                    _, vs = plsc.sort_key_val(key, val)
                    cval[pl.ds(cnt, 16)] = vs

                return cnt + pc

            cnt = lax.fori_loop(0, HALF // 16, scan_step, jnp.int32(0),
                                unroll=4)

            def fill(k, _):
                cval[pl.ds(cnt + k * 16, 16)] = trash_packed
                return 0

            lax.fori_loop(0, 8, fill, 0)
            return cnt

        def do_table(idx_ref, t_hbm, out_hbm):
            ccp = pltpu.async_copy(
                t_hbm.at[:, pl.ds(dma_lo, CW)], chunk, csem)
            hi = jnp.minimum(lo + CW, TAIL0)
            c0 = scan_round(idx_ref, 0, lo, hi, col_lo)
            ccp.wait()
            run_batches(out_hbm, c0)
            c1 = scan_round(idx_ref, HALF, lo, hi, col_lo)
            run_batches(out_hbm, c1)

        do_table(uidx, ut_h, ou_h)
        do_table(pidx, pt_h, op_h)

        def do_tail(idx_ref, tail_hbm, out_hbm):
            pltpu.async_copy(tail_hbm, chunk.at[:, pl.ds(0, 128)], csem).wait()

            def scan_step(j, cnt):
                base = wid * 128 + j * 16
                v16 = idx_ref[pl.ds(base, 16)]
                m = v16 >= TAIL0
                pc = plsc.all_reduce_population_count(m)[0]

                @pl.when(pc > 0)
                def _():
                    key = 1 - m.astype(jnp.int32)
                    val = jnp.where(m, posP + base * PACK + (v16 - TAIL0),
                                    trash_packed)
                    _, vs = plsc.sort_key_val(key, val)
                    cval[pl.ds(cnt, 16)] = vs

                return cnt + pc

            cnt = lax.fori_loop(0, 8, scan_step, jnp.int32(0))

            @pl.when(cnt > 0)
            def _():
                def fill(k, _):
                    cval[pl.ds(cnt + k * 16, 16)] = trash_packed
                    return 0

                lax.fori_loop(0, 8, fill, 0)
                run_batches(out_hbm, jnp.minimum(cnt, 128))

        do_tail(uidx, tu_h, ou_h)
        do_tail(pidx, tp_h, op_h)

    return sc_gather


def _mlp_body(ue_ref, pe_ref, w1_ref, b1_ref, w2_ref, b2_ref,
              w3_ref, b3_ref, out_ref, etu_ref, etp_ref):
    xu = ue_ref[...][:, :D]
    xp = pe_ref[...][:, :D]
    etu_ref[...] = xu.T
    etp_ref[...] = xp.T
    w1u = w1_ref[...][:D]
    w1p = w1_ref[...][D:]
    h = jnp.dot(xu, w1u, preferred_element_type=jnp.float32)
    h = h + jnp.dot(xp, w1p, preferred_element_type=jnp.float32)
    h = jnp.maximum(h + b1_ref[...], 0.0)
    h = jnp.dot(h, w2_ref[...], preferred_element_type=jnp.float32)
    h = jnp.maximum(h + b2_ref[...], 0.0)
    r = jnp.dot(h, w3_ref[...], preferred_element_type=jnp.float32)
    out_ref[...] = r + b3_ref[...]


def kernel(user_id, product_id, user_table, product_table, W1, b1, W2, b2, W3, b3):
    uT = user_table.T
    pT = product_table.T
    tail_u = jnp.pad(lax.slice(uT, (0, TAIL0), (D, V)),
                     ((0, 0), (0, 128 - (V - TAIL0))))
    tail_p = jnp.pad(lax.slice(pT, (0, TAIL0), (D, V)),
                     ((0, 0), (0, 128 - (V - TAIL0))))

    info = plsc.get_sparse_core_info()
    sc_gather = _make_sc_gather(info.num_cores, info.num_subcores)
    ou, op = sc_gather(user_id, product_id, uT, pT, tail_u, tail_p)

    b1r = b1.reshape(1, -1)
    b2r = b2.reshape(1, -1)
    b3r = b3.reshape(1, 1)

    rating, embTu, embTp = pl.pallas_call(
        _mlp_body,
        grid=(1,),
        in_specs=[
            pl.BlockSpec((B, 128), lambda i: (0, 0)),
            pl.BlockSpec((B, 128), lambda i: (0, 0)),
            pl.BlockSpec(W1.shape, lambda i: (0, 0)),
            pl.BlockSpec(b1r.shape, lambda i: (0, 0)),
            pl.BlockSpec(W2.shape, lambda i: (0, 0)),
            pl.BlockSpec(b2r.shape, lambda i: (0, 0)),
            pl.BlockSpec(W3.shape, lambda i: (0, 0)),
            pl.BlockSpec(b3r.shape, lambda i: (0, 0)),
        ],
        out_specs=[
            pl.BlockSpec((B, 1), lambda i: (0, 0)),
            pl.BlockSpec((D, B), lambda i: (0, 0)),
            pl.BlockSpec((D, B), lambda i: (0, 0)),
        ],
        out_shape=[
            jax.ShapeDtypeStruct((B, 1), jnp.float32),
            jax.ShapeDtypeStruct((D, B), jnp.float32),
            jax.ShapeDtypeStruct((D, B), jnp.float32),
        ],
    )(ou, op, W1, b1r, W2, b2r, W3, b3r)

    return (embTu.T, embTp.T, rating)


# speedup vs baseline: 1.0716x; 1.0716x over previous
import functools

import jax
import jax.numpy as jnp
from jax import lax
from jax.experimental import pallas as pl
from jax.experimental.pallas import tpu as pltpu
from jax.experimental.pallas import tpu_sc as plsc

B = 4096
V = 100001
D = 32
CW = 3200
TAIL0 = 99968
LASTLO = 96768
OUTROWS = B + 128
PACK = 4096
HALF = 2048
MAXB = HALF // 128 + 1


def _make_sc_gather(num_cores, num_subcores):
    NW = num_cores * num_subcores
    assert (NW - 1) * CW < TAIL0 <= LASTLO + CW
    mesh = plsc.VectorSubcoreMesh(core_axis_name="c", subcore_axis_name="s")

    @functools.partial(
        pl.kernel,
        mesh=mesh,
        compiler_params=pltpu.CompilerParams(
            use_tc_tiling_on_sc=True, needs_layout_passes=False),
        out_type=[
            jax.ShapeDtypeStruct((OUTROWS, 128), jnp.float32),
            jax.ShapeDtypeStruct((OUTROWS, 128), jnp.float32),
        ],
        scratch_types=[
            pltpu.VMEM((D, CW), jnp.float32),
            pltpu.VMEM((B,), jnp.int32),
            pltpu.VMEM((B,), jnp.int32),
            pltpu.VMEM((HALF + 144,), jnp.int32),
            pltpu.VMEM((128,), jnp.int32),
            pltpu.VMEM((128, 128), jnp.float32),
            pltpu.SemaphoreType.DMA,
            pltpu.SemaphoreType.DMA,
            pltpu.SemaphoreType.DMA,
        ],
    )
    def sc_gather(uid_h, pid_h, ut_h, pt_h, tu_h, tp_h, ou_h, op_h,
                  chunk, uidx, pidx, cval, cpos, staging,
                  csem, isem, ssem):
        cix = lax.axis_index("c")
        six = lax.axis_index("s")
        wid = six * num_cores + cix
        lo = wid * CW
        col_lo = jnp.minimum(lo, LASTLO)
        dma_lo = pl.multiple_of(col_lo, 128)

        iota16 = lax.iota(jnp.int32, 16)
        posP = iota16 * PACK
        trash16 = 4096 + ((iota16 * 8 + wid) & 127)
        trash_packed = trash16 * PACK

        icp_u = pltpu.async_copy(uid_h, uidx, isem)
        icp_p = pltpu.async_copy(pid_h, pidx, isem)
        icp_u.wait()
        icp_p.wait()

        fconst = [jnp.full((16,), f, jnp.int32) for f in range(D)]

        def run_batches(out_hbm, cnt):
            def batch(b, _):
                @pl.when(b * 128 < cnt)
                def _():
                    def per_group(g, _):
                        val16 = cval[pl.ds(b * 128 + g * 16, 16)]
                        pos16 = lax.shift_right_logical(val16, 12)
                        cpos[pl.ds(g * 16, 16)] = pos16
                        q16 = val16 & (PACK - 1)
                        ent16 = g * 16 + iota16
                        for f in range(D):
                            r = plsc.load_gather(chunk, [fconst[f], q16])
                            plsc.store_scatter(staging, [ent16, fconst[f]], r)
                        return 0

                    lax.fori_loop(0, 8, per_group, 0)
                    pltpu.async_copy(staging, out_hbm.at[cpos], ssem).wait()

                return 0

            lax.fori_loop(0, MAXB, batch, 0)

        def scan_round(idx_ref, pos0, in_lo, in_hi, sub):
            spill16 = jnp.full((16,), HALF + 128, jnp.int32)

            def scan_step(j, cntv):
                v16 = idx_ref[pl.ds(pos0 + j * 16, 16)]
                m = (v16 >= in_lo) & (v16 < in_hi)
                mi = m.astype(jnp.int32)
                cs = plsc.cumsum(mi)
                addr = jnp.where(m, cntv + cs - 1, spill16)
                val = (iota16 + pos0 + j * 16) * PACK + (v16 - sub)
                plsc.store_scatter(cval, [addr], val)
                return cntv + plsc.all_reduce_population_count(m)

            cntv = lax.fori_loop(0, HALF // 16, scan_step,
                                 jnp.zeros((16,), jnp.int32), unroll=4)
            cnt = cntv[0]

            def fill(k, _):
                cval[pl.ds(cnt + k * 16, 16)] = trash_packed
                return 0

            lax.fori_loop(0, 8, fill, 0)
            return cnt

        def do_table(idx_ref, t_hbm, out_hbm):
            ccp = pltpu.async_copy(
                t_hbm.at[:, pl.ds(dma_lo, CW)], chunk, csem)
            hi = jnp.minimum(lo + CW, TAIL0)
            c0 = scan_round(idx_ref, 0, lo, hi, col_lo)
            ccp.wait()
            run_batches(out_hbm, c0)
            c1 = scan_round(idx_ref, HALF, lo, hi, col_lo)
            run_batches(out_hbm, c1)

        do_table(uidx, ut_h, ou_h)
        do_table(pidx, pt_h, op_h)

        def do_tail(idx_ref, tail_hbm, out_hbm):
            pltpu.async_copy(tail_hbm, chunk.at[:, pl.ds(0, 128)], csem).wait()

            def scan_step(j, cnt):
                base = wid * 128 + j * 16
                v16 = idx_ref[pl.ds(base, 16)]
                m = v16 >= TAIL0
                pc = plsc.all_reduce_population_count(m)[0]

                @pl.when(pc > 0)
                def _():
                    key = 1 - m.astype(jnp.int32)
                    val = jnp.where(m, posP + base * PACK + (v16 - TAIL0),
                                    trash_packed)
                    _, vs = plsc.sort_key_val(key, val)
                    cval[pl.ds(cnt, 16)] = vs

                return cnt + pc

            cnt = lax.fori_loop(0, 8, scan_step, jnp.int32(0))

            @pl.when(cnt > 0)
            def _():
                def fill(k, _):
                    cval[pl.ds(cnt + k * 16, 16)] = trash_packed
                    return 0

                lax.fori_loop(0, 8, fill, 0)
                run_batches(out_hbm, jnp.minimum(cnt, 128))

        do_tail(uidx, tu_h, ou_h)
        do_tail(pidx, tp_h, op_h)

    return sc_gather


def _mlp_body(ue_ref, pe_ref, w1_ref, b1_ref, w2_ref, b2_ref,
              w3_ref, b3_ref, out_ref, etu_ref, etp_ref):
    xu = ue_ref[...][:, :D]
    xp = pe_ref[...][:, :D]
    etu_ref[...] = xu.T
    etp_ref[...] = xp.T
    w1u = w1_ref[...][:D]
    w1p = w1_ref[...][D:]
    h = jnp.dot(xu, w1u, preferred_element_type=jnp.float32)
    h = h + jnp.dot(xp, w1p, preferred_element_type=jnp.float32)
    h = jnp.maximum(h + b1_ref[...], 0.0)
    h = jnp.dot(h, w2_ref[...], preferred_element_type=jnp.float32)
    h = jnp.maximum(h + b2_ref[...], 0.0)
    r = jnp.dot(h, w3_ref[...], preferred_element_type=jnp.float32)
    out_ref[...] = r + b3_ref[...]


def kernel(user_id, product_id, user_table, product_table, W1, b1, W2, b2, W3, b3):
    uT = user_table.T
    pT = product_table.T
    tail_u = jnp.pad(lax.slice(uT, (0, TAIL0), (D, V)),
                     ((0, 0), (0, 128 - (V - TAIL0))))
    tail_p = jnp.pad(lax.slice(pT, (0, TAIL0), (D, V)),
                     ((0, 0), (0, 128 - (V - TAIL0))))

    info = plsc.get_sparse_core_info()
    sc_gather = _make_sc_gather(info.num_cores, info.num_subcores)
    ou, op = sc_gather(user_id, product_id, uT, pT, tail_u, tail_p)

    b1r = b1.reshape(1, -1)
    b2r = b2.reshape(1, -1)
    b3r = b3.reshape(1, 1)

    rating, embTu, embTp = pl.pallas_call(
        _mlp_body,
        grid=(1,),
        in_specs=[
            pl.BlockSpec((B, 128), lambda i: (0, 0)),
            pl.BlockSpec((B, 128), lambda i: (0, 0)),
            pl.BlockSpec(W1.shape, lambda i: (0, 0)),
            pl.BlockSpec(b1r.shape, lambda i: (0, 0)),
            pl.BlockSpec(W2.shape, lambda i: (0, 0)),
            pl.BlockSpec(b2r.shape, lambda i: (0, 0)),
            pl.BlockSpec(W3.shape, lambda i: (0, 0)),
            pl.BlockSpec(b3r.shape, lambda i: (0, 0)),
        ],
        out_specs=[
            pl.BlockSpec((B, 1), lambda i: (0, 0)),
            pl.BlockSpec((D, B), lambda i: (0, 0)),
            pl.BlockSpec((D, B), lambda i: (0, 0)),
        ],
        out_shape=[
            jax.ShapeDtypeStruct((B, 1), jnp.float32),
            jax.ShapeDtypeStruct((D, B), jnp.float32),
            jax.ShapeDtypeStruct((D, B), jnp.float32),
        ],
    )(ou, op, W1, b1r, W2, b2r, W3, b3r)

    return (embTu.T, embTp.T, rating)


# speedup vs baseline: 1.0948x; 1.0217x over previous
import functools

import jax
import jax.numpy as jnp
from jax import lax
from jax.experimental import pallas as pl
from jax.experimental.pallas import tpu as pltpu
from jax.experimental.pallas import tpu_sc as plsc

B = 4096
V = 100001
D = 32
CW = 3200
TAIL0 = 99968
LASTLO = 96768
OUTROWS = B + 128
PACK = 4096
HALF = 2048
MAXB = HALF // 128 + 1


def _make_sc_gather(num_cores, num_subcores):
    NW = num_cores * num_subcores
    assert (NW - 1) * CW < TAIL0 <= LASTLO + CW
    mesh = plsc.VectorSubcoreMesh(core_axis_name="c", subcore_axis_name="s")

    @functools.partial(
        pl.kernel,
        mesh=mesh,
        compiler_params=pltpu.CompilerParams(
            use_tc_tiling_on_sc=True, needs_layout_passes=False),
        out_type=[
            jax.ShapeDtypeStruct((OUTROWS, 128), jnp.float32),
            jax.ShapeDtypeStruct((OUTROWS, 128), jnp.float32),
        ],
        scratch_types=[
            pltpu.VMEM((D, CW), jnp.float32),
            pltpu.VMEM((B,), jnp.int32),
            pltpu.VMEM((B,), jnp.int32),
            pltpu.VMEM((HALF + 144,), jnp.int32),
            pltpu.VMEM((128,), jnp.int32),
            pltpu.VMEM((128, 128), jnp.float32),
            pltpu.SemaphoreType.DMA,
            pltpu.SemaphoreType.DMA,
            pltpu.SemaphoreType.DMA,
        ],
    )
    def sc_gather(uid_h, pid_h, ut_h, pt_h, tu_h, tp_h, ou_h, op_h,
                  chunk, uidx, pidx, cval, cpos, staging,
                  csem, isem, ssem):
        cix = lax.axis_index("c")
        six = lax.axis_index("s")
        wid = six * num_cores + cix
        lo = wid * CW
        col_lo = jnp.minimum(lo, LASTLO)
        dma_lo = pl.multiple_of(col_lo, 128)

        iota16 = lax.iota(jnp.int32, 16)
        posP = iota16 * PACK
        trash16 = 4096 + ((iota16 * 8 + wid) & 127)
        trash_packed = trash16 * PACK

        icp_u = pltpu.async_copy(uid_h, uidx, isem)
        icp_p = pltpu.async_copy(pid_h, pidx, isem)
        icp_u.wait()
        icp_p.wait()

        fconst = [jnp.full((16,), f, jnp.int32) for f in range(D)]

        def run_batches(out_hbm, cnt):
            def batch(b, _):
                @pl.when(b * 128 < cnt)
                def _():
                    def per_group(g, _):
                        val16 = cval[pl.ds(b * 128 + g * 16, 16)]
                        pos16 = lax.shift_right_logical(val16, 12)
                        cpos[pl.ds(g * 16, 16)] = pos16
                        q16 = val16 & (PACK - 1)
                        ent16 = g * 16 + iota16
                        for f in range(D):
                            r = plsc.load_gather(chunk, [fconst[f], q16])
                            plsc.store_scatter(staging, [ent16, fconst[f]], r)
                        return 0

                    lax.fori_loop(0, 8, per_group, 0)
                    pltpu.async_copy(staging, out_hbm.at[cpos], ssem).wait()

                return 0

            lax.fori_loop(0, MAXB, batch, 0)

        def scan_round(idx_ref, pos0, in_lo, in_hi, sub):
            def scan_step(j, cnt):
                v16 = idx_ref[pl.ds(pos0 + j * 16, 16)]
                m = (v16 >= in_lo) & (v16 < in_hi)
                val = (iota16 + pos0 + j * 16) * PACK + (v16 - sub)
                plsc.store_compressed(cval.at[pl.ds(cnt, 16)], val, mask=m)
                return cnt + plsc.all_reduce_population_count(m)[0]

            cnt = lax.fori_loop(0, HALF // 16, scan_step, jnp.int32(0),
                                unroll=8)

            def fill(k, _):
                cval[pl.ds(cnt + k * 16, 16)] = trash_packed
                return 0

            lax.fori_loop(0, 8, fill, 0)
            return cnt

        def do_table(idx_ref, t_hbm, out_hbm):
            ccp = pltpu.async_copy(
                t_hbm.at[:, pl.ds(dma_lo, CW)], chunk, csem)
            hi = jnp.minimum(lo + CW, TAIL0)
            c0 = scan_round(idx_ref, 0, lo, hi, col_lo)
            ccp.wait()
            run_batches(out_hbm, c0)
            c1 = scan_round(idx_ref, HALF, lo, hi, col_lo)
            run_batches(out_hbm, c1)

        do_table(uidx, ut_h, ou_h)
        do_table(pidx, pt_h, op_h)

        def do_tail(idx_ref, tail_hbm, out_hbm):
            pltpu.async_copy(tail_hbm, chunk.at[:, pl.ds(0, 128)], csem).wait()

            def scan_step(j, cnt):
                base = wid * 128 + j * 16
                v16 = idx_ref[pl.ds(base, 16)]
                m = v16 >= TAIL0
                val = (iota16 + base) * PACK + (v16 - TAIL0)
                plsc.store_compressed(cval.at[pl.ds(cnt, 16)], val, mask=m)
                return cnt + plsc.all_reduce_population_count(m)[0]

            cnt = lax.fori_loop(0, 8, scan_step, jnp.int32(0))

            @pl.when(cnt > 0)
            def _():
                def fill(k, _):
                    cval[pl.ds(cnt + k * 16, 16)] = trash_packed
                    return 0

                lax.fori_loop(0, 8, fill, 0)
                run_batches(out_hbm, jnp.minimum(cnt, 128))

        do_tail(uidx, tu_h, ou_h)
        do_tail(pidx, tp_h, op_h)

    return sc_gather


def _mlp_body(ue_ref, pe_ref, w1_ref, b1_ref, w2_ref, b2_ref,
              w3_ref, b3_ref, out_ref, etu_ref, etp_ref):
    xu = ue_ref[...][:, :D]
    xp = pe_ref[...][:, :D]
    etu_ref[...] = xu.T
    etp_ref[...] = xp.T
    w1u = w1_ref[...][:D]
    w1p = w1_ref[...][D:]
    h = jnp.dot(xu, w1u, preferred_element_type=jnp.float32)
    h = h + jnp.dot(xp, w1p, preferred_element_type=jnp.float32)
    h = jnp.maximum(h + b1_ref[...], 0.0)
    h = jnp.dot(h, w2_ref[...], preferred_element_type=jnp.float32)
    h = jnp.maximum(h + b2_ref[...], 0.0)
    r = jnp.dot(h, w3_ref[...], preferred_element_type=jnp.float32)
    out_ref[...] = (r + b3_ref[...]).T


def kernel(user_id, product_id, user_table, product_table, W1, b1, W2, b2, W3, b3):
    uT = user_table.T
    pT = product_table.T
    tail_u = jnp.pad(lax.slice(uT, (0, TAIL0), (D, V)),
                     ((0, 0), (0, 128 - (V - TAIL0))))
    tail_p = jnp.pad(lax.slice(pT, (0, TAIL0), (D, V)),
                     ((0, 0), (0, 128 - (V - TAIL0))))

    info = plsc.get_sparse_core_info()
    sc_gather = _make_sc_gather(info.num_cores, info.num_subcores)
    ou, op = sc_gather(user_id, product_id, uT, pT, tail_u, tail_p)

    b1r = b1.reshape(1, -1)
    b2r = b2.reshape(1, -1)
    b3r = b3.reshape(1, 1)

    rating, embTu, embTp = pl.pallas_call(
        _mlp_body,
        grid=(1,),
        in_specs=[
            pl.BlockSpec((B, 128), lambda i: (0, 0)),
            pl.BlockSpec((B, 128), lambda i: (0, 0)),
            pl.BlockSpec(W1.shape, lambda i: (0, 0)),
            pl.BlockSpec(b1r.shape, lambda i: (0, 0)),
            pl.BlockSpec(W2.shape, lambda i: (0, 0)),
            pl.BlockSpec(b2r.shape, lambda i: (0, 0)),
            pl.BlockSpec(W3.shape, lambda i: (0, 0)),
            pl.BlockSpec(b3r.shape, lambda i: (0, 0)),
        ],
        out_specs=[
            pl.BlockSpec((1, B), lambda i: (0, 0)),
            pl.BlockSpec((D, B), lambda i: (0, 0)),
            pl.BlockSpec((D, B), lambda i: (0, 0)),
        ],
        out_shape=[
            jax.ShapeDtypeStruct((1, B), jnp.float32),
            jax.ShapeDtypeStruct((D, B), jnp.float32),
            jax.ShapeDtypeStruct((D, B), jnp.float32),
        ],
    )(ou, op, W1, b1r, W2, b2r, W3, b3r)

    return (embTu.T, embTp.T, rating.T)


# speedup vs baseline: 1.1668x; 1.0657x over previous
import functools

import jax
import jax.numpy as jnp
from jax import lax
from jax.experimental import pallas as pl
from jax.experimental.pallas import tpu as pltpu
from jax.experimental.pallas import tpu_sc as plsc

B = 4096
V = 100001
D = 32
CW = 3200
TAIL0 = 99968
LASTLO = 96768
OUTROWS = B + 128
PACK = 4096
HALF = 2048
MAXB = HALF // 128 + 1


def _make_sc_gather(num_cores, num_subcores):
    NW = num_cores * num_subcores
    assert (NW - 1) * CW < TAIL0 <= LASTLO + CW
    mesh = plsc.VectorSubcoreMesh(core_axis_name="c", subcore_axis_name="s")

    @functools.partial(
        pl.kernel,
        mesh=mesh,
        compiler_params=pltpu.CompilerParams(
            use_tc_tiling_on_sc=True, needs_layout_passes=False),
        out_type=[
            jax.ShapeDtypeStruct((OUTROWS, 128), jnp.float32),
            jax.ShapeDtypeStruct((OUTROWS, 128), jnp.float32),
        ],
        scratch_types=[
            pltpu.VMEM((D, CW), jnp.float32),
            pltpu.VMEM((B,), jnp.int32),
            pltpu.VMEM((B,), jnp.int32),
            pltpu.VMEM((HALF + 144,), jnp.int32),
            pltpu.VMEM((128,), jnp.int32),
            pltpu.VMEM((128, 128), jnp.float32),
            pltpu.SemaphoreType.DMA,
            pltpu.SemaphoreType.DMA,
            pltpu.SemaphoreType.DMA,
        ],
    )
    def sc_gather(uid_h, pid_h, ut_h, pt_h, tu_h, tp_h, ou_h, op_h,
                  chunk, uidx, pidx, cval, cpos, staging,
                  csem, isem, ssem):
        cix = lax.axis_index("c")
        six = lax.axis_index("s")
        wid = six * num_cores + cix
        lo = wid * CW
        col_lo = jnp.minimum(lo, LASTLO)
        dma_lo = pl.multiple_of(col_lo, 128)

        iota16 = lax.iota(jnp.int32, 16)
        posP = iota16 * PACK
        trash16 = 4096 + ((iota16 * 8 + wid) & 127)
        trash_packed = trash16 * PACK

        icp_u = pltpu.async_copy(uid_h, uidx, isem)
        icp_p = pltpu.async_copy(pid_h, pidx, isem)
        icp_u.wait()
        icp_p.wait()

        fconst = [jnp.full((16,), f, jnp.int32) for f in range(D)]

        def run_batches(out_hbm, cnt):
            def batch(b, _):
                @pl.when(b * 128 < cnt)
                def _():
                    def per_group(g, _):
                        val16 = cval[pl.ds(b * 128 + g * 16, 16)]
                        pos16 = lax.shift_right_logical(val16, 12)
                        cpos[pl.ds(g * 16, 16)] = pos16
                        q16 = val16 & (PACK - 1)
                        ent16 = g * 16 + iota16
                        for f in range(D):
                            r = plsc.load_gather(chunk, [fconst[f], q16])
                            plsc.store_scatter(staging, [ent16, fconst[f]], r)
                        return 0

                    rem = cnt - b * 128
                    ng = jnp.minimum((rem + 15) >> 4, 8)
                    lax.fori_loop(0, ng, per_group, 0)

                    def fill_pos(g, _):
                        cpos[pl.ds(g * 16, 16)] = trash16
                        return 0

                    lax.fori_loop(ng, 8, fill_pos, 0)
                    pltpu.async_copy(staging, out_hbm.at[cpos], ssem).wait()

                return 0

            lax.fori_loop(0, MAXB, batch, 0)

        def scan_round(idx_ref, pos0, in_lo, in_hi, sub):
            def scan_step(j, cnt):
                v16 = idx_ref[pl.ds(pos0 + j * 16, 16)]
                m = (v16 >= in_lo) & (v16 < in_hi)
                val = (iota16 + pos0 + j * 16) * PACK + (v16 - sub)
                plsc.store_compressed(cval.at[pl.ds(cnt, 16)], val, mask=m)
                return cnt + plsc.all_reduce_population_count(m)[0]

            cnt = lax.fori_loop(0, HALF // 16, scan_step, jnp.int32(0),
                                unroll=8)

            def fill(k, _):
                cval[pl.ds(cnt + k * 16, 16)] = trash_packed
                return 0

            lax.fori_loop(0, 8, fill, 0)
            return cnt

        def do_table(idx_ref, t_hbm, out_hbm):
            ccp = pltpu.async_copy(
                t_hbm.at[:, pl.ds(dma_lo, CW)], chunk, csem)
            hi = jnp.minimum(lo + CW, TAIL0)
            c0 = scan_round(idx_ref, 0, lo, hi, col_lo)
            ccp.wait()
            run_batches(out_hbm, c0)
            c1 = scan_round(idx_ref, HALF, lo, hi, col_lo)
            run_batches(out_hbm, c1)

        do_table(uidx, ut_h, ou_h)
        do_table(pidx, pt_h, op_h)

        def do_tail(idx_ref, tail_hbm, out_hbm):
            pltpu.async_copy(tail_hbm, chunk.at[:, pl.ds(0, 128)], csem).wait()

            def scan_step(j, cnt):
                base = wid * 128 + j * 16
                v16 = idx_ref[pl.ds(base, 16)]
                m = v16 >= TAIL0
                val = (iota16 + base) * PACK + (v16 - TAIL0)
                plsc.store_compressed(cval.at[pl.ds(cnt, 16)], val, mask=m)
                return cnt + plsc.all_reduce_population_count(m)[0]

            cnt = lax.fori_loop(0, 8, scan_step, jnp.int32(0))

            @pl.when(cnt > 0)
            def _():
                def fill(k, _):
                    cval[pl.ds(cnt + k * 16, 16)] = trash_packed
                    return 0

                lax.fori_loop(0, 8, fill, 0)
                run_batches(out_hbm, jnp.minimum(cnt, 128))

        do_tail(uidx, tu_h, ou_h)
        do_tail(pidx, tp_h, op_h)

    return sc_gather


def _mlp_body(ue_ref, pe_ref, w1_ref, b1_ref, w2_ref, b2_ref,
              w3_ref, b3_ref, out_ref, etu_ref, etp_ref):
    xu = ue_ref[...][:, :D]
    xp = pe_ref[...][:, :D]
    etu_ref[...] = xu.T
    etp_ref[...] = xp.T
    w1u = w1_ref[...][:D]
    w1p = w1_ref[...][D:]
    h = jnp.dot(xu, w1u, preferred_element_type=jnp.float32)
    h = h + jnp.dot(xp, w1p, preferred_element_type=jnp.float32)
    h = jnp.maximum(h + b1_ref[...], 0.0)
    h = jnp.dot(h, w2_ref[...], preferred_element_type=jnp.float32)
    h = jnp.maximum(h + b2_ref[...], 0.0)
    r = jnp.dot(h, w3_ref[...], preferred_element_type=jnp.float32)
    out_ref[...] = (r + b3_ref[...]).T


def kernel(user_id, product_id, user_table, product_table, W1, b1, W2, b2, W3, b3):
    uT = user_table.T
    pT = product_table.T
    tail_u = jnp.pad(lax.slice(uT, (0, TAIL0), (D, V)),
                     ((0, 0), (0, 128 - (V - TAIL0))))
    tail_p = jnp.pad(lax.slice(pT, (0, TAIL0), (D, V)),
                     ((0, 0), (0, 128 - (V - TAIL0))))

    info = plsc.get_sparse_core_info()
    sc_gather = _make_sc_gather(info.num_cores, info.num_subcores)
    ou, op = sc_gather(user_id, product_id, uT, pT, tail_u, tail_p)

    b1r = b1.reshape(1, -1)
    b2r = b2.reshape(1, -1)
    b3r = b3.reshape(1, 1)

    rating, embTu, embTp = pl.pallas_call(
        _mlp_body,
        grid=(1,),
        in_specs=[
            pl.BlockSpec((B, 128), lambda i: (0, 0)),
            pl.BlockSpec((B, 128), lambda i: (0, 0)),
            pl.BlockSpec(W1.shape, lambda i: (0, 0)),
            pl.BlockSpec(b1r.shape, lambda i: (0, 0)),
            pl.BlockSpec(W2.shape, lambda i: (0, 0)),
            pl.BlockSpec(b2r.shape, lambda i: (0, 0)),
            pl.BlockSpec(W3.shape, lambda i: (0, 0)),
            pl.BlockSpec(b3r.shape, lambda i: (0, 0)),
        ],
        out_specs=[
            pl.BlockSpec((1, B), lambda i: (0, 0)),
            pl.BlockSpec((D, B), lambda i: (0, 0)),
            pl.BlockSpec((D, B), lambda i: (0, 0)),
        ],
        out_shape=[
            jax.ShapeDtypeStruct((1, B), jnp.float32),
            jax.ShapeDtypeStruct((D, B), jnp.float32),
            jax.ShapeDtypeStruct((D, B), jnp.float32),
        ],
    )(ou, op, W1, b1r, W2, b2r, W3, b3r)

    return (embTu.T, embTp.T, rating.T)
